# baseline (speedup 1.0000x reference)
import jax
import jax.numpy as jnp
from jax.experimental import pallas as pl
from jax.experimental.pallas import tpu as pltpu

_BATCH = 4
_N_CTX = 4096
_D_MODEL = 2048
_N_EXPERTS = 64

_CHUNK = 512
_NBUF = 8
_N_CHUNKS = (_BATCH * _N_CTX) // _CHUNK


def _router_kernel(x_hbm, w_ref, b_ref, out_w_ref, out_e_ref, x_buf, sem):
    def copy(chunk, slot):
        return pltpu.make_async_copy(
            x_hbm.at[pl.ds(chunk * _CHUNK, _CHUNK), :],
            x_buf.at[slot],
            sem.at[slot],
        )

    for s in range(_NBUF - 1):
        copy(s, s).start()

    def body(i, carry):
        for k in range(_NBUF):
            c = i * _NBUF + k
            copy(c, k).wait()

            @pl.when(c + _NBUF - 1 < _N_CHUNKS)
            def _():
                copy(c + _NBUF - 1, (k + _NBUF - 1) % _NBUF).start()

            xb = x_buf[k]
            logits = jax.lax.dot_general(
                w_ref[...], xb,
                dimension_numbers=(((1,), (1,)), ((), ())),
                preferred_element_type=jnp.float32,
            )
            logits = logits + b_ref[...]
            m = jnp.max(logits, axis=0, keepdims=True)
            eidx = jax.lax.broadcasted_iota(jnp.int32, logits.shape, 0)
            idx = jnp.min(jnp.where(logits == m, eidx, _N_EXPERTS),
                          axis=0, keepdims=True)
            s_ = jnp.sum(jnp.exp(logits - m), axis=0, keepdims=True)
            out_w_ref[pl.ds(c, 1), 0, :] = 1.0 / s_
            out_e_ref[pl.ds(c, 1), 0, :] = idx
        return carry

    jax.lax.fori_loop(0, _N_CHUNKS // _NBUF, body, 0)


@jax.jit
def kernel(x, W, b):
    tokens = _BATCH * _N_CTX
    xf = x.reshape(tokens, _D_MODEL)
    b2 = b.reshape(_N_EXPERTS, 1)

    weights, experts = pl.pallas_call(
        _router_kernel,
        grid=(1,),
        in_specs=[
            pl.BlockSpec(memory_space=pltpu.MemorySpace.HBM),
            pl.BlockSpec((_N_EXPERTS, _D_MODEL), lambda i: (0, 0)),
            pl.BlockSpec((_N_EXPERTS, 1), lambda i: (0, 0)),
        ],
        out_specs=[
            pl.BlockSpec((_N_CHUNKS, 1, _CHUNK), lambda i: (0, 0, 0)),
            pl.BlockSpec((_N_CHUNKS, 1, _CHUNK), lambda i: (0, 0, 0)),
        ],
        out_shape=[
            jax.ShapeDtypeStruct((_N_CHUNKS, 1, _CHUNK), jnp.float32),
            jax.ShapeDtypeStruct((_N_CHUNKS, 1, _CHUNK), jnp.int32),
        ],
        scratch_shapes=[
            pltpu.VMEM((_NBUF, _CHUNK, _D_MODEL), jnp.float32),
            pltpu.SemaphoreType.DMA((_NBUF,)),
        ],
        compiler_params=pltpu.CompilerParams(
            dimension_semantics=("arbitrary",),
        ),
    )(xf, W, b2)

    weights = weights.reshape(_BATCH, _N_CTX)
    experts = experts.reshape(_BATCH, _N_CTX)
    return (weights, experts)


# speedup vs baseline: 1.0259x; 1.0259x over previous
import jax
import jax.numpy as jnp
from jax.experimental import pallas as pl
from jax.experimental.pallas import tpu as pltpu

_BATCH = 4
_N_CTX = 4096
_D_MODEL = 2048
_N_EXPERTS = 64

_TILE = 1024
_N_STEPS = (_BATCH * _N_CTX) // (2 * _TILE)


def _one_tile(xb, w_ref, b_ref):
    logits = jax.lax.dot_general(
        w_ref[...], xb,
        dimension_numbers=(((1,), (1,)), ((), ())),
        preferred_element_type=jnp.float32,
    )
    logits = logits + b_ref[...]
    m = jnp.max(logits, axis=0, keepdims=True)
    eidx = jax.lax.broadcasted_iota(jnp.int32, logits.shape, 0)
    idx = jnp.min(jnp.where(logits == m, eidx, _N_EXPERTS),
                  axis=0, keepdims=True)
    s = jnp.sum(jnp.exp(logits - m), axis=0, keepdims=True)
    return 1.0 / s, idx


def _router_kernel(xa_ref, xb_ref, w_ref, b_ref, out_w_ref, out_e_ref):
    w0, i0 = _one_tile(xa_ref[...], w_ref, b_ref)
    w1, i1 = _one_tile(xb_ref[...], w_ref, b_ref)
    out_w_ref[0:1, 0, :] = w0
    out_w_ref[1:2, 0, :] = w1
    out_e_ref[0:1, 0, :] = i0
    out_e_ref[1:2, 0, :] = i1


@jax.jit
def kernel(x, W, b):
    tokens = _BATCH * _N_CTX
    xf = x.reshape(tokens, _D_MODEL)
    b2 = b.reshape(_N_EXPERTS, 1)

    weights, experts = pl.pallas_call(
        _router_kernel,
        grid=(_N_STEPS,),
        in_specs=[
            pl.BlockSpec((_TILE, _D_MODEL), lambda i: (2 * i, 0)),
            pl.BlockSpec((_TILE, _D_MODEL), lambda i: (2 * i + 1, 0)),
            pl.BlockSpec((_N_EXPERTS, _D_MODEL), lambda i: (0, 0)),
            pl.BlockSpec((_N_EXPERTS, 1), lambda i: (0, 0)),
        ],
        out_specs=[
            pl.BlockSpec((2, 1, _TILE), lambda i: (i, 0, 0)),
            pl.BlockSpec((2, 1, _TILE), lambda i: (i, 0, 0)),
        ],
        out_shape=[
            jax.ShapeDtypeStruct((2 * _N_STEPS, 1, _TILE), jnp.float32),
            jax.ShapeDtypeStruct((2 * _N_STEPS, 1, _TILE), jnp.int32),
        ],
        compiler_params=pltpu.CompilerParams(
            dimension_semantics=("arbitrary",),
        ),
    )(xf, xf, W, b2)

    weights = weights.reshape(_BATCH, _N_CTX)
    experts = experts.reshape(_BATCH, _N_CTX)
    return (weights, experts)


# speedup vs baseline: 1.0320x; 1.0060x over previous
import jax
import jax.numpy as jnp
from jax.experimental import pallas as pl
from jax.experimental.pallas import tpu as pltpu

_BATCH = 4
_N_CTX = 4096
_D_MODEL = 2048
_N_EXPERTS = 64

_TILE = 1024


def _router_kernel(x_ref, w_ref, b_ref, out_w_ref, out_e_ref):
    xb = x_ref[...]
    logits = jax.lax.dot_general(
        w_ref[...], xb,
        dimension_numbers=(((1,), (1,)), ((), ())),
        preferred_element_type=jnp.float32,
    )
    logits = logits + b_ref[...]
    m = jnp.max(logits, axis=0, keepdims=True)
    eidx = jax.lax.broadcasted_iota(jnp.int32, logits.shape, 0)
    idx = jnp.min(jnp.where(logits == m, eidx, _N_EXPERTS), axis=0)
    s = jnp.sum(jnp.exp(logits - m), axis=0)
    out_w_ref[0, 0, :] = 1.0 / s
    out_e_ref[0, 0, :] = idx


@jax.jit
def kernel(x, W, b):
    tokens = _BATCH * _N_CTX
    n_tiles = tokens // _TILE
    xf = x.reshape(tokens, _D_MODEL)
    b2 = b.reshape(_N_EXPERTS, 1)

    grid = (n_tiles,)
    weights, experts = pl.pallas_call(
        _router_kernel,
        grid=grid,
        in_specs=[
            pl.BlockSpec((_TILE, _D_MODEL), lambda i: (i, 0)),
            pl.BlockSpec((_N_EXPERTS, _D_MODEL), lambda i: (0, 0)),
            pl.BlockSpec((_N_EXPERTS, 1), lambda i: (0, 0)),
        ],
        out_specs=[
            pl.BlockSpec((1, 1, _TILE), lambda i: (i, 0, 0)),
            pl.BlockSpec((1, 1, _TILE), lambda i: (i, 0, 0)),
        ],
        out_shape=[
            jax.ShapeDtypeStruct((n_tiles, 1, _TILE), jnp.float32),
            jax.ShapeDtypeStruct((n_tiles, 1, _TILE), jnp.int32),
        ],
        compiler_params=pltpu.CompilerParams(
            dimension_semantics=("parallel",),
        ),
    )(xf, W, b2)

    weights = weights.reshape(_BATCH, _N_CTX)
    experts = experts.reshape(_BATCH, _N_CTX)
    return (weights, experts)
